# baseline (device time: 116727 ns/iter reference)
import jax
import jax.numpy as jnp
from jax import lax
from jax.experimental import pallas as pl
from jax.experimental.pallas import tpu as pltpu

N_CHUNK = 8


def kernel(O, Wo):
    B, S, H, D = O.shape
    K = H * D
    N = Wo.shape[1]
    S_half = S // 2
    S_chunk = S_half // N_CHUNK

    O2 = O.reshape(B, S, K)

    def body(o_ref, w_ref, out_ref,
             w_vmem, o_peer, o_mine, send_buf, recv_buf, out_stage,
             w_sem, o_peer_sems, o_mine_sem,
             send_sems, recv_sems, store_sems):
        my_x = lax.axis_index("x")
        my_y = lax.axis_index("y")
        my_z = lax.axis_index("z")
        peer = (1 - my_x, my_y, my_z)

        my_start = my_x * S_half
        peer_start = (1 - my_x) * S_half

        w_load = pltpu.make_async_copy(w_ref, w_vmem, w_sem)
        w_load.start()
        o_loads = []
        for c in range(N_CHUNK):
            ld = pltpu.make_async_copy(
                o_ref.at[:, pl.ds(peer_start + c * S_chunk, S_chunk), :],
                o_peer.at[c],
                o_peer_sems.at[c],
            )
            ld.start()
            o_loads.append(ld)
        mine_load = pltpu.make_async_copy(
            o_ref.at[:, pl.ds(my_start, S_half), :], o_mine, o_mine_sem,
        )
        mine_load.start()

        barrier_sem = pltpu.get_barrier_semaphore()
        pl.semaphore_signal(
            barrier_sem, inc=1, device_id=peer,
            device_id_type=pl.DeviceIdType.MESH,
        )
        pl.semaphore_wait(barrier_sem, 1)

        w_load.wait()
        w = w_vmem[:, :]

        def chunk_rdma(c):
            return pltpu.make_async_remote_copy(
                src_ref=send_buf.at[c],
                dst_ref=recv_buf.at[c],
                send_sem=send_sems.at[c],
                recv_sem=recv_sems.at[c],
                device_id=peer,
                device_id_type=pl.DeviceIdType.MESH,
            )

        rdmas = []
        for c in range(N_CHUNK):
            o_loads[c].wait()
            send_buf[c, :, :, :] = lax.dot_general(
                o_peer[c], w, (((2,), (0,)), ((), ())),
                preferred_element_type=jnp.float32,
            )
            rdma = chunk_rdma(c)
            rdma.start()
            rdmas.append(rdma)

        mine_load.wait()
        out_stage[:, :, :] = lax.dot_general(
            o_mine[:, :, :], w, (((2,), (0,)), ((), ())),
            preferred_element_type=jnp.float32,
        )

        stores = []
        for c in range(N_CHUNK):
            rdmas[c].wait_recv()
            rows = pl.ds(c * S_chunk, S_chunk)
            out_stage[:, rows, :] += recv_buf[c, :, :, :]
            st = pltpu.make_async_copy(
                out_stage.at[:, rows, :], out_ref.at[:, rows, :],
                store_sems.at[c],
            )
            st.start()
            stores.append(st)

        for st in stores:
            st.wait()
        for c in range(N_CHUNK):
            rdmas[c].wait_send()

    return pl.pallas_call(
        body,
        out_shape=jax.ShapeDtypeStruct((B, S_half, N), jnp.float32),
        in_specs=[
            pl.BlockSpec(memory_space=pl.ANY),
            pl.BlockSpec(memory_space=pl.ANY),
        ],
        out_specs=pl.BlockSpec(memory_space=pl.ANY),
        scratch_shapes=[
            pltpu.VMEM((K, N), jnp.float32),
            pltpu.VMEM((N_CHUNK, B, S_chunk, K), jnp.float32),
            pltpu.VMEM((B, S_half, K), jnp.float32),
            pltpu.VMEM((N_CHUNK, B, S_chunk, N), jnp.float32),
            pltpu.VMEM((N_CHUNK, B, S_chunk, N), jnp.float32),
            pltpu.VMEM((B, S_half, N), jnp.float32),
            pltpu.SemaphoreType.DMA,
            pltpu.SemaphoreType.DMA((N_CHUNK,)),
            pltpu.SemaphoreType.DMA,
            pltpu.SemaphoreType.DMA((N_CHUNK,)),
            pltpu.SemaphoreType.DMA((N_CHUNK,)),
            pltpu.SemaphoreType.DMA((N_CHUNK,)),
        ],
        compiler_params=pltpu.CompilerParams(
            collective_id=0,
            vmem_limit_bytes=64 * 1024 * 1024,
        ),
    )(O2, Wo)
